# baseline (speedup 1.0000x reference)
import functools

import jax
import jax.numpy as jnp
from jax import lax
from jax.experimental import pallas as pl
from jax.experimental.pallas import tpu as pltpu
from jax.experimental.pallas import tpu_sc as plsc

N = 10000
IN = 256
H = 512
L = 256
E = 160000

NC, NS = 2, 16
NW = NC * NS
EB = 128
EPT = 5120
BPT = EPT // EB
E_PAD = NW * EPT
EBP = 64
EPT_S = E_PAD // NS
BPT_S = EPT_S // EBP
NA = 10112
RPA = NA // NS
NP = 10240
RPT = NP // NS
BN = 640
GRID = NP // BN

def _mesh():
    return plsc.VectorSubcoreMesh(core_axis_name="c", subcore_axis_name="s",
                                  num_cores=NC, num_subcores=NS)


def _deg_body(dst2_hbm, ones_hbm, zeros_hbm, deg_hbm, idx_v, ones_v, acc_sh):
    c = lax.axis_index("c")
    s = lax.axis_index("s")
    wid = s * NC + c
    pltpu.sync_copy(ones_hbm, ones_v)
    pltpu.sync_copy(zeros_hbm.at[pl.ds(s * RPT, RPT)],
                    acc_sh.at[pl.ds(s * RPT, RPT)])
    pltpu.sync_copy(dst2_hbm.at[pl.ds(wid * BPT, BPT)], idx_v)
    plsc.subcore_barrier()

    def body(b, carry):
        pltpu.sync_copy(ones_v, acc_sh.at[idx_v.at[b]], add=True)
        return carry

    lax.fori_loop(0, BPT, body, 0)
    plsc.subcore_barrier()
    pltpu.sync_copy(acc_sh.at[pl.ds(s * RPT, RPT)],
                    deg_hbm.at[c].at[pl.ds(s * RPT, RPT)])


def _sc_degrees(dst2):
    ones = jnp.ones((EB, 128), jnp.float32)
    zeros = jnp.zeros((NP, 128), jnp.float32)
    return pl.kernel(
        _deg_body,
        out_type=jax.ShapeDtypeStruct((NC, NP, 128), jnp.float32),
        mesh=_mesh(),
        scratch_types=[
            pltpu.VMEM((BPT, EB), jnp.int32),
            pltpu.VMEM((EB, 128), jnp.float32),
            pltpu.VMEM_SHARED((NP, 128), jnp.float32),
        ],
    )(dst2, ones, zeros)


def _prop_body(nqh, u_hbm, src1_hbm, dst1_hbm, out_hbm,
               sidx_v, dia0, dia1, dia2, dia3, r0, r1, r2, r3, acc_sh,
               sg0, sg1, sg2, sg3, si0, si1, si2, si3):
    c = lax.axis_index("c")
    s = lax.axis_index("s")
    pltpu.sync_copy(src1_hbm.at[pl.ds(s * EPT_S, EPT_S)], sidx_v)
    dia = (dia0, dia1, dia2, dia3)
    rows = (r0, r1, r2, r3)
    sg = (sg0, sg1, sg2, sg3)
    si = (si0, si1, si2, si3)

    def gidx(b):
        return sidx_v.at[pl.ds(b * EBP, EBP)]

    for j in range(nqh):
        q = c * nqh + j
        uq = u_hbm.at[q]
        pltpu.sync_copy(uq.at[pl.ds(s * RPA, RPA)],
                        acc_sh.at[pl.ds(s * RPA, RPA)])
        plsc.subcore_barrier()

        for k in range(4):
            pltpu.async_copy(uq.at[gidx(k)], rows[k], sg[k])
            pltpu.async_copy(
                dst1_hbm.at[pl.ds(s * EPT_S + k * EBP, EBP)], dia[k], si[k])

        def body(i, carry):
            for k in range(4):
                b = i * 4 + k
                pltpu.make_async_copy(uq.at[gidx(b)], rows[k], sg[k]).wait()
                pltpu.make_async_copy(
                    dst1_hbm.at[pl.ds(0, EBP)], dia[k], si[k]).wait()
                pltpu.sync_copy(rows[k], acc_sh.at[dia[k]], add=True)

                @pl.when(b + 4 < BPT_S)
                def _():
                    pltpu.async_copy(uq.at[gidx(b + 4)], rows[k], sg[k])
                    pltpu.async_copy(
                        dst1_hbm.at[pl.ds(s * EPT_S + (b + 4) * EBP, EBP)],
                        dia[k], si[k])
            return carry

        lax.fori_loop(0, BPT_S // 4, body, 0)
        plsc.subcore_barrier()
        pltpu.sync_copy(acc_sh.at[pl.ds(s * RPA, RPA)],
                        out_hbm.at[q].at[pl.ds(s * RPA, RPA)])
        plsc.subcore_barrier()


def _sc_prop(u, src1, dst1):
    nq = u.shape[0]
    return pl.kernel(
        functools.partial(_prop_body, nq // NC),
        out_type=jax.ShapeDtypeStruct((nq, NP, 128), jnp.float32),
        mesh=_mesh(),
        scratch_types=[
            pltpu.VMEM((EPT_S,), jnp.int32),
            pltpu.VMEM((EBP,), jnp.int32),
            pltpu.VMEM((EBP,), jnp.int32),
            pltpu.VMEM((EBP,), jnp.int32),
            pltpu.VMEM((EBP,), jnp.int32),
            pltpu.VMEM((EBP, 128), jnp.float32),
            pltpu.VMEM((EBP, 128), jnp.float32),
            pltpu.VMEM((EBP, 128), jnp.float32),
            pltpu.VMEM((EBP, 128), jnp.float32),
            pltpu.VMEM_SHARED((NA, 128), jnp.float32),
        ] + [pltpu.SemaphoreType.DMA] * 8,
    )(u, src1, dst1)


def _row_spec(shape):
    nd = len(shape)
    if nd == 2:
        return pl.BlockSpec((BN, shape[1]), lambda i: (i, 0))
    return pl.BlockSpec((shape[0], BN, shape[2]), lambda i: (0, i, 0))


def _full_spec(shape):
    return pl.BlockSpec(shape, lambda i: tuple(0 for _ in shape))


def _bdot(a, w):
    return jnp.dot(a.astype(jnp.bfloat16), w.astype(jnp.bfloat16),
                   preferred_element_type=jnp.float32)


def _ln_relu(v, g, n):
    mu = jnp.mean(v, axis=-1, keepdims=True)
    var = jnp.mean((v - mu) * (v - mu), axis=-1, keepdims=True)
    h = (v - mu) * lax.rsqrt(var + 1e-5) * g + n
    return jnp.maximum(h, 0.0)


def _kin_body(tf_ref, x_ref, deg_ref, wt1_ref, bt1_ref, wt2_ref, bt2_ref,
              wc1_ref, u_ref, dinv_ref):
    d = deg_ref[0, :, 0] + deg_ref[1, :, 0]
    dinv = lax.rsqrt(1.0 + d)[:, None]
    dinv_ref[...] = dinv
    a = tf_ref[...] * wt1_ref[...] + bt1_ref[...]
    a = a * jax.nn.sigmoid(a)
    temb = _bdot(a, wt2_ref[...]) + bt2_ref[...]
    u = _bdot(x_ref[...], wc1_ref[0:IN, :]) + _bdot(temb, wc1_ref[IN:2 * IN, :])
    u = u * dinv
    for q in range(4):
        u_ref[q] = u[:, q * 128:(q + 1) * 128]


def _tc_in(tf, x, deg2, wt1, bt1, wt2, bt2, wc1):
    return pl.pallas_call(
        _kin_body,
        grid=(GRID,),
        in_specs=[_row_spec(tf.shape), _row_spec(x.shape), _row_spec(deg2.shape),
                  _full_spec(wt1.shape), _full_spec(bt1.shape),
                  _full_spec(wt2.shape), _full_spec(bt2.shape),
                  _full_spec(wc1.shape)],
        out_specs=[_row_spec((4, NP, 128)), _row_spec((NP, 1))],
        out_shape=[jax.ShapeDtypeStruct((4, NP, 128), jnp.float32),
                   jax.ShapeDtypeStruct((NP, 1), jnp.float32)],
    )(tf, x, deg2, wt1, bt1, wt2, bt2, wc1)


def _kmid_body(nq_in, nq_out, has_skip, emit_h, *refs):
    s_ref, dinv_ref, b_ref, g_ref, n_ref, w_ref = refs[:6]
    refs = refs[6:]
    if has_skip:
        skip_ref, refs = refs[0], refs[1:]
    u_ref = refs[0]
    v = jnp.concatenate([s_ref[q] for q in range(nq_in)], axis=-1)
    v = v * dinv_ref[...] + b_ref[...]
    h = _ln_relu(v, g_ref[...], n_ref[...])
    if has_skip:
        h = h + skip_ref[...]
    if emit_h:
        refs[1][...] = h
    u = _bdot(h, w_ref[...]) * dinv_ref[...]
    for q in range(nq_out):
        u_ref[q] = u[:, q * 128:(q + 1) * 128]


def _tc_mid(s, dinv, b, g, n, w, skip=None, emit_h=False):
    nq_in = s.shape[0]
    nq_out = w.shape[1] // 128
    din = nq_in * 128
    ins = [s, dinv, b, g, n, w]
    in_specs = [_row_spec(s.shape), _row_spec(dinv.shape),
                _full_spec(b.shape), _full_spec(g.shape), _full_spec(n.shape),
                _full_spec(w.shape)]
    if skip is not None:
        ins.append(skip)
        in_specs.append(_row_spec(skip.shape))
    out_specs = [_row_spec((nq_out, NP, 128))]
    out_shape = [jax.ShapeDtypeStruct((nq_out, NP, 128), jnp.float32)]
    if emit_h:
        out_specs.append(_row_spec((NP, din)))
        out_shape.append(jax.ShapeDtypeStruct((NP, din), jnp.float32))
    res = pl.pallas_call(
        functools.partial(_kmid_body, nq_in, nq_out, skip is not None, emit_h),
        grid=(GRID,),
        in_specs=in_specs,
        out_specs=out_specs,
        out_shape=out_shape,
    )(*ins)
    return res if emit_h else res[0]


def _kout_body(s_ref, dinv_ref, b_ref, g_ref, n_ref, wm1_ref, bm1_ref,
               wm2_ref, bm2_ref, o_ref):
    v = jnp.concatenate([s_ref[0], s_ref[1]], axis=-1)
    v = v * dinv_ref[...] + b_ref[...]
    h = _ln_relu(v, g_ref[...], n_ref[...])
    z = jnp.maximum(_bdot(h, wm1_ref[...]) + bm1_ref[...], 0.0)
    o_ref[...] = _bdot(z, wm2_ref[...]) + bm2_ref[...]


def _tc_out(s, dinv, b, g, n, wm1, bm1, wm2, bm2):
    return pl.pallas_call(
        _kout_body,
        grid=(GRID,),
        in_specs=[_row_spec(s.shape), _row_spec(dinv.shape),
                  _full_spec(b.shape), _full_spec(g.shape), _full_spec(n.shape),
                  _full_spec(wm1.shape), _full_spec(bm1.shape),
                  _full_spec(wm2.shape), _full_spec(bm2.shape)],
        out_specs=_row_spec((NP, IN)),
        out_shape=jax.ShapeDtypeStruct((NP, IN), jnp.float32),
    )(s, dinv, b, g, n, wm1, bm1, wm2, bm2)


def kernel(x, edge_index, t, Wt1, bt1, Wt2, bt2, Wc1, bc1, g1, n1, Wc2, bc2,
           g2, n2, Wd1, bd1, g3, n3, Wd2, bd2, g4, n4, Wm1, bm1, Wm2, bm2):
    src = edge_index[0]
    dst = edge_index[1]
    pad = E_PAD - E
    src1 = jnp.concatenate([src, jnp.zeros((pad,), src.dtype)])
    dst1 = jnp.concatenate([dst, jnp.full((pad,), N, dst.dtype)])
    dst2 = dst1.reshape(E_PAD // EB, EB)
    tf = jnp.pad(t.reshape(-1, 1).astype(jnp.float32), ((0, NP - N), (0, 0)))
    x = jnp.pad(x, ((0, NP - N), (0, 0)))

    r = lambda a: a.reshape(1, -1)

    deg2 = _sc_degrees(dst2)
    U1, dinv = _tc_in(tf, x, deg2, Wt1, r(bt1), Wt2, r(bt2), Wc1)
    S1 = _sc_prop(U1, src1, dst1)
    U2, h1 = _tc_mid(S1, dinv, r(bc1), r(g1), r(n1), Wc2, emit_h=True)
    S2 = _sc_prop(U2, src1, dst1)
    U3 = _tc_mid(S2, dinv, r(bc2), r(g2), r(n2), Wd1, skip=h1)
    S3 = _sc_prop(U3, src1, dst1)
    U4 = _tc_mid(S3, dinv, r(bd1), r(g3), r(n3), Wd2)
    S4 = _sc_prop(U4, src1, dst1)
    out = _tc_out(S4, dinv, r(bd2), r(g4), r(n4), Wm1, r(bm1), Wm2, r(bm2))
    return out[:N]
